# Initial kernel scaffold; baseline (speedup 1.0000x reference)
import jax
import jax.numpy as jnp
from jax.experimental import pallas as pl


def kernel(x, mask):
    raise NotImplementedError("write your pallas kernel here")


# speedup vs baseline: 1.6648x; 1.6648x over previous
import functools

import jax
import jax.numpy as jnp
from jax import lax
from jax.experimental import pallas as pl
from jax.experimental.pallas import tpu as pltpu
from jax.experimental.pallas import tpu_sc as plsc

R, C = 128, 32768
LANES = 16
CHUNK = 4096
NCHUNK = C // CHUNK


def _build_sc_kernel():
    mesh = plsc.VectorSubcoreMesh(core_axis_name="c", subcore_axis_name="s")
    info = plsc.get_sparse_core_info()
    nc, ns = info.num_cores, info.num_subcores
    nw = nc * ns
    rows_per_w = R // nw

    @functools.partial(
        pl.kernel,
        mesh=mesh,
        compiler_params=pltpu.CompilerParams(needs_layout_passes=False),
        out_type=jax.ShapeDtypeStruct((R, C), jnp.float32),
        scratch_types=[
            pltpu.VMEM((CHUNK,), jnp.float32),
            pltpu.VMEM((CHUNK,), jnp.float32),
            pltpu.VMEM((CHUNK,), jnp.float32),
        ],
    )
    def k(x_hbm, m_hbm, out_hbm, x_v, m_v, o_v):
        wid = lax.axis_index("s") * nc + lax.axis_index("c")
        for r in range(rows_per_w):
            row = wid * rows_per_w + r

            def chunk_body(cidx, carry):
                base = cidx * CHUNK
                pltpu.sync_copy(x_hbm.at[row, pl.ds(base, CHUNK)], x_v)
                pltpu.sync_copy(m_hbm.at[row, pl.ds(base, CHUNK)], m_v)

                def vec_body(i, cy):
                    off = i * LANES
                    xm = x_v[pl.ds(off, LANES)] * m_v[pl.ds(off, LANES)]
                    o_v[pl.ds(off, LANES)] = jnp.cumsum(xm) + cy
                    return cy + jnp.sum(xm)

                carry = lax.fori_loop(0, CHUNK // LANES, vec_body, carry)
                pltpu.sync_copy(o_v, out_hbm.at[row, pl.ds(base, CHUNK)])
                return carry

            lax.fori_loop(0, NCHUNK, chunk_body, jnp.float32(0.0))

    return k


_sc_kernel = _build_sc_kernel()


@jax.jit
def kernel(x, mask):
    return _sc_kernel(x, mask.astype(jnp.float32))


# speedup vs baseline: 1.8794x; 1.1289x over previous
import functools

import jax
import jax.numpy as jnp
from jax import lax
from jax.experimental import pallas as pl
from jax.experimental.pallas import tpu as pltpu
from jax.experimental.pallas import tpu_sc as plsc

R, C = 128, 32768
LANES = 16
CHUNK = 4096
NCHUNK = C // CHUNK


def _build_sc_kernel():
    mesh = plsc.VectorSubcoreMesh(core_axis_name="c", subcore_axis_name="s")
    info = plsc.get_sparse_core_info()
    nc, ns = info.num_cores, info.num_subcores
    nw = nc * ns
    rows_per_w = R // nw

    @functools.partial(
        pl.kernel,
        mesh=mesh,
        compiler_params=pltpu.CompilerParams(needs_layout_passes=False),
        out_type=jax.ShapeDtypeStruct((R, C), jnp.float32),
        scratch_types=[
            pltpu.VMEM((CHUNK,), jnp.float32),
            pltpu.VMEM((CHUNK,), jnp.float32),
            pltpu.VMEM((CHUNK,), jnp.float32),
        ],
    )
    def k(x_hbm, m_hbm, out_hbm, x_v, m_v, o_v):
        wid = lax.axis_index("s") * nc + lax.axis_index("c")
        for r in range(rows_per_w):
            row = wid * rows_per_w + r

            def chunk_body(cidx, carry):
                base = cidx * CHUNK
                pltpu.sync_copy(x_hbm.at[row, pl.ds(base, CHUNK)], x_v)
                pltpu.sync_copy(m_hbm.at[row, pl.ds(base, CHUNK)], m_v)

                @plsc.parallel_loop(0, CHUNK // LANES, unroll=8, carry=carry)
                def vec_body(i, cy):
                    off = i * LANES
                    xm = x_v[pl.ds(off, LANES)] * m_v[pl.ds(off, LANES)]
                    s = jnp.cumsum(xm)
                    o_v[pl.ds(off, LANES)] = s + cy
                    return cy + s[15]

                pltpu.sync_copy(o_v, out_hbm.at[row, pl.ds(base, CHUNK)])
                return vec_body

            lax.fori_loop(0, NCHUNK, chunk_body, jnp.float32(0.0))

    return k


_sc_kernel = _build_sc_kernel()


@jax.jit
def kernel(x, mask):
    return _sc_kernel(x, mask.astype(jnp.float32))


# speedup vs baseline: 3.7780x; 2.0102x over previous
import functools

import jax
import jax.numpy as jnp
from jax import lax
from jax.experimental import pallas as pl
from jax.experimental.pallas import tpu as pltpu
from jax.experimental.pallas import tpu_sc as plsc

R, C = 128, 32768
LANES = 16
CHUNK = 8192
NCHUNK = C // CHUNK


def _build_sc_kernel():
    mesh = plsc.VectorSubcoreMesh(core_axis_name="c", subcore_axis_name="s")
    info = plsc.get_sparse_core_info()
    nc, ns = info.num_cores, info.num_subcores
    nw = nc * ns
    rpw = R // nw
    T = rpw * NCHUNK

    @functools.partial(
        pl.kernel,
        mesh=mesh,
        compiler_params=pltpu.CompilerParams(needs_layout_passes=False),
        out_type=jax.ShapeDtypeStruct((R, C), jnp.float32),
        scratch_types=[
            pltpu.VMEM((CHUNK,), jnp.float32),
            pltpu.VMEM((CHUNK,), jnp.float32),
            pltpu.VMEM((CHUNK,), jnp.float32),
            pltpu.VMEM((CHUNK,), jnp.float32),
            pltpu.VMEM((CHUNK,), jnp.float32),
            pltpu.VMEM((CHUNK,), jnp.float32),
            pltpu.SemaphoreType.DMA,
            pltpu.SemaphoreType.DMA,
            pltpu.SemaphoreType.DMA,
            pltpu.SemaphoreType.DMA,
            pltpu.SemaphoreType.DMA,
            pltpu.SemaphoreType.DMA,
        ],
    )
    def k(x_hbm, m_hbm, out_hbm, x_v0, x_v1, m_v0, m_v1, o_v0, o_v1,
          sx0, sx1, sm0, sm1, so0, so1):
        x_v, m_v, o_v = (x_v0, x_v1), (m_v0, m_v1), (o_v0, o_v1)
        sx, sm, so = (sx0, sx1), (sm0, sm1), (so0, so1)
        wid = lax.axis_index("s") * nc + lax.axis_index("c")
        row0 = wid * rpw

        def in_slices(t):
            row = row0 + t // NCHUNK
            base = (t % NCHUNK) * CHUNK
            return row, base

        def start_load(t, b):
            row, base = in_slices(t)
            pltpu.async_copy(x_hbm.at[row, pl.ds(base, CHUNK)], x_v[b], sx[b])
            pltpu.async_copy(m_hbm.at[row, pl.ds(base, CHUNK)], m_v[b], sm[b])

        def wait_load(t, b):
            row, base = in_slices(t)
            pltpu.make_async_copy(x_hbm.at[row, pl.ds(base, CHUNK)], x_v[b], sx[b]).wait()
            pltpu.make_async_copy(m_hbm.at[row, pl.ds(base, CHUNK)], m_v[b], sm[b]).wait()

        def wait_store(t, b):
            row, base = in_slices(t)
            pltpu.make_async_copy(o_v[b], out_hbm.at[row, pl.ds(base, CHUNK)], so[b]).wait()

        start_load(0, 0)

        def pair_body(p, carry):
            for b in range(2):
                t = 2 * p + b
                wait_load(t, b)

                @pl.when(t + 1 < T)
                def _():
                    start_load(t + 1, 1 - b)

                @pl.when(t >= 2)
                def _():
                    wait_store(t, b)

                carry = jnp.where(t % NCHUNK == 0, jnp.float32(0.0), carry)
                xb, mb, ob = x_v[b], m_v[b], o_v[b]

                @plsc.parallel_loop(0, CHUNK // LANES, unroll=8, carry=carry)
                def vec_body(i, cy):
                    off = i * LANES
                    xm = xb[pl.ds(off, LANES)] * mb[pl.ds(off, LANES)]
                    s = jnp.cumsum(xm)
                    ob[pl.ds(off, LANES)] = s + cy
                    return cy + s[15]

                carry = vec_body
                row, base = in_slices(t)
                pltpu.async_copy(o_v[b], out_hbm.at[row, pl.ds(base, CHUNK)], so[b])
            return carry

        lax.fori_loop(0, T // 2, pair_body, jnp.float32(0.0))
        wait_store(T - 2, 0)
        wait_store(T - 1, 1)

    return k


_sc_kernel = _build_sc_kernel()


@jax.jit
def kernel(x, mask):
    return _sc_kernel(x, mask.astype(jnp.float32))
